# baseline (speedup 1.0000x reference)
import functools

import jax
import jax.numpy as jnp
from jax import lax
from jax.experimental import pallas as pl
from jax.experimental.pallas import tpu as pltpu
from jax.experimental.pallas import tpu_sc as plsc


def _ffnet_body(x_h, w3_h, b3_h, w4_h, b4_h, w5_h, b5_h, out_h,
                x_v, w3_v, b3_v, w4_v, b4_v, w5_v, b5_v, out_v, sem):
    c = lax.axis_index("c")
    s = lax.axis_index("s")

    @pl.when(jnp.logical_and(c == 0, s == 0))
    def _():
        cps = [
            pltpu.async_copy(x_h, x_v.at[0:3], sem),
            pltpu.async_copy(w3_h, w3_v.at[0:2], sem),
            pltpu.async_copy(b3_h, b3_v.at[0:1], sem),
            pltpu.async_copy(w4_h, w4_v.at[0:3], sem),
            pltpu.async_copy(b4_h, b4_v.at[0:1], sem),
            pltpu.async_copy(w5_h, w5_v.at[0:2], sem),
            pltpu.async_copy(b5_h, b5_v.at[0:1], sem),
        ]
        for cp in cps:
            cp.wait()

        xv = x_v[...]
        w3v = w3_v[...]
        b3v = b3_v[...]
        w4v = w4_v[...]
        b4v = b4_v[...]
        w5v = w5_v[...]
        b5v = b5_v[...]

        def bc(val):
            return jnp.full((16,), val, jnp.float32)

        one = jnp.float32(1.0)
        x0 = bc(xv[0])
        x1 = bc(xv[1])
        z3 = x0 * bc(w3v[0]) + x1 * bc(w3v[1]) + bc(b3v[0])
        s3 = one / (one + jnp.exp(-z3))
        z4 = (x0 * bc(w4v[0]) + x1 * bc(w4v[1])
              + s3 * bc(w4v[2]) + bc(b4v[0]))
        s4 = one / (one + jnp.exp(-z4))
        z5 = s3 * bc(w5v[0]) + s4 * bc(w5v[1]) + bc(b5v[0])
        out_v[...] = one / (one + jnp.exp(-z5))
        pltpu.sync_copy(out_v.at[0:1], out_h)


_ffnet = functools.partial(
    pl.kernel,
    out_type=jax.ShapeDtypeStruct((1,), jnp.float32),
    mesh=plsc.VectorSubcoreMesh(core_axis_name="c", subcore_axis_name="s",
                                num_cores=1, num_subcores=1),
    scratch_types=[
        pltpu.VMEM((16,), jnp.float32),
        pltpu.VMEM((16,), jnp.float32),
        pltpu.VMEM((16,), jnp.float32),
        pltpu.VMEM((16,), jnp.float32),
        pltpu.VMEM((16,), jnp.float32),
        pltpu.VMEM((16,), jnp.float32),
        pltpu.VMEM((16,), jnp.float32),
        pltpu.VMEM((16,), jnp.float32),
        pltpu.SemaphoreType.DMA,
    ],
)(_ffnet_body)


def kernel(x, W3, b3, W4, b4, W5, b5):
    out = _ffnet(x.reshape(3), W3.reshape(2), b3, W4.reshape(3), b4,
                 W5.reshape(2), b5)
    return out.reshape(1, 1)


# speedup vs baseline: 5.7511x; 5.7511x over previous
import jax
import jax.numpy as jnp
from jax.experimental import pallas as pl
from jax.experimental.pallas import tpu as pltpu


def _ffnet_body(x_ref, w3_ref, b3_ref, w4_ref, b4_ref, w5_ref, b5_ref,
                out_ref):
    one = jnp.float32(1.0)
    x = x_ref[...]
    w3 = w3_ref[...]
    w4 = w4_ref[...]
    w5 = w5_ref[...]
    x0 = x[:, 0:1]
    x1 = x[:, 1:2]
    z3 = x0 * w3[:, 0:1] + x1 * w3[:, 1:2] + b3_ref[0]
    s3 = one / (one + jnp.exp(-z3))
    z4 = x0 * w4[:, 0:1] + x1 * w4[:, 1:2] + s3 * w4[:, 2:3] + b4_ref[0]
    s4 = one / (one + jnp.exp(-z4))
    z5 = s3 * w5[:, 0:1] + s4 * w5[:, 1:2] + b5_ref[0]
    out_ref[...] = one / (one + jnp.exp(-z5))


_vmem = pl.BlockSpec(memory_space=pltpu.VMEM)
_smem = pl.BlockSpec(memory_space=pltpu.SMEM)

_ffnet = pl.pallas_call(
    _ffnet_body,
    out_shape=jax.ShapeDtypeStruct((1, 1), jnp.float32),
    in_specs=[_vmem, _vmem, _smem, _vmem, _smem, _vmem, _smem],
    out_specs=_vmem,
)


def kernel(x, W3, b3, W4, b4, W5, b5):
    return _ffnet(x, W3, b3, W4, b4, W5, b5)
